# Initial kernel scaffold; baseline (speedup 1.0000x reference)
import jax
import jax.numpy as jnp
from jax.experimental import pallas as pl


def kernel(x, edge_index, Wl1, bl1, Wr1, Wl2, bl2, Wr2, Wl3, bl3, Wr3):
    raise NotImplementedError("write your pallas kernel here")


# speedup vs baseline: 9.7842x; 9.7842x over previous
import functools

import jax
import jax.numpy as jnp
from jax import lax
from jax.experimental import pallas as pl
from jax.experimental.pallas import tpu as pltpu
from jax.experimental.pallas import tpu_sc as plsc

_N = 10000
_D = 128
_E = 320000
_NSUB = 16
_NCORE = 2
_CHUNK = 125
_NCHUNK = 80
_RPS = _N // _NSUB


def _make_agg(width):
    mesh = plsc.VectorSubcoreMesh(core_axis_name="c", subcore_axis_name="s")

    def body(edges, table, zeros, out, acc, idx_v, rows_v, sem0, sem1):
        c = lax.axis_index("c")
        s = lax.axis_index("s")
        w = c * _NSUB + s
        base = s * _RPS
        pltpu.sync_copy(zeros.at[pl.ds(base, _RPS)], acc.at[pl.ds(base, _RPS)])
        pltpu.sync_copy(edges.at[w, 0], idx_v.at[0])
        pltpu.async_copy(table.at[idx_v.at[0, 0]], rows_v.at[0], sem0)
        plsc.subcore_barrier()

        @pl.loop(0, _NCHUNK, step=2)
        def _(j):
            pltpu.sync_copy(edges.at[w, j + 1], idx_v.at[1])
            pltpu.async_copy(table.at[idx_v.at[1, 0]], rows_v.at[1], sem1)
            pltpu.make_async_copy(
                table.at[pl.ds(0, _CHUNK)], rows_v.at[0], sem0).wait()
            pltpu.sync_copy(rows_v.at[0], acc.at[idx_v.at[0, 1]], add=True)

            @pl.when(j + 2 < _NCHUNK)
            def _():
                pltpu.sync_copy(edges.at[w, j + 2], idx_v.at[0])
                pltpu.async_copy(table.at[idx_v.at[0, 0]], rows_v.at[0], sem0)

            pltpu.make_async_copy(
                table.at[pl.ds(0, _CHUNK)], rows_v.at[1], sem1).wait()
            pltpu.sync_copy(rows_v.at[1], acc.at[idx_v.at[1, 1]], add=True)

        plsc.subcore_barrier()
        pltpu.sync_copy(acc.at[pl.ds(base, _RPS)],
                        out.at[c, pl.ds(base, _RPS)])

    return pl.kernel(
        body,
        out_type=jax.ShapeDtypeStruct((_NCORE, _N, width), jnp.float32),
        mesh=mesh,
        compiler_params=pltpu.CompilerParams(use_tc_tiling_on_sc=False),
        scratch_types=[
            pltpu.VMEM_SHARED((_N, width), jnp.float32),
            pltpu.VMEM((2, 2, _CHUNK), jnp.int32),
            pltpu.VMEM((2, _CHUNK, width), jnp.float32),
            pltpu.SemaphoreType.DMA,
            pltpu.SemaphoreType.DMA,
        ],
    )


_agg_l1 = _make_agg(_D + 16)
_agg = _make_agg(_D)


def _layer1_body(p_ref, x_ref, wl_ref, bl_ref, wr_ref, h_ref, inv_ref):
    msum = p_ref[0, :, :_D] + p_ref[1, :, :_D]
    cnt = p_ref[0, :, _D:_D + 1] + p_ref[1, :, _D:_D + 1]
    inv = 1.0 / jnp.maximum(cnt, 1.0)
    mean = msum * inv
    h = jnp.dot(mean, wl_ref[...], preferred_element_type=jnp.float32)
    h = h + bl_ref[...] + jnp.dot(x_ref[...], wr_ref[...],
                                  preferred_element_type=jnp.float32)
    h_ref[...] = jnp.maximum(h, 0.0)
    inv_ref[...] = inv


_layer1_tc = pl.pallas_call(
    _layer1_body,
    out_shape=(
        jax.ShapeDtypeStruct((_N, _D), jnp.float32),
        jax.ShapeDtypeStruct((_N, 1), jnp.float32),
    ),
)


def _make_layer23(relu):
    def body(p_ref, inv_ref, x_ref, wl_ref, bl_ref, wr_ref, o_ref):
        mean = (p_ref[0] + p_ref[1]) * inv_ref[...]
        h = jnp.dot(mean, wl_ref[...], preferred_element_type=jnp.float32)
        h = h + bl_ref[...] + jnp.dot(x_ref[...], wr_ref[...],
                                      preferred_element_type=jnp.float32)
        o_ref[...] = jnp.maximum(h, 0.0) if relu else h

    return pl.pallas_call(
        body, out_shape=jax.ShapeDtypeStruct((_N, _D), jnp.float32))


_layer2_tc = _make_layer23(True)
_layer3_tc = _make_layer23(False)


def kernel(x, edge_index, Wl1, bl1, Wr1, Wl2, bl2, Wr2, Wl3, bl3, Wr3):
    ei = edge_index.astype(jnp.int32).reshape(2, _NCORE * _NSUB, _NCHUNK, _CHUNK)
    edges = jnp.stack([ei[0], ei[1]], axis=2)
    xa = jnp.concatenate(
        [x, jnp.ones((_N, 1), jnp.float32), jnp.zeros((_N, 15), jnp.float32)],
        axis=1)
    z_l1 = jnp.zeros((_N, _D + 16), jnp.float32)
    z = jnp.zeros((_N, _D), jnp.float32)
    bl1r = bl1.reshape(1, _D)
    bl2r = bl2.reshape(1, _D)
    bl3r = bl3.reshape(1, _D)

    p1 = _agg_l1(edges, xa, z_l1)
    h1, inv = _layer1_tc(p1, x, Wl1, bl1r, Wr1)
    p2 = _agg(edges, h1, z)
    h2 = _layer2_tc(p2, inv, h1, Wl2, bl2r, Wr2)
    p3 = _agg(edges, h2, z)
    return _layer3_tc(p3, inv, h2, Wl3, bl3r, Wr3)


# speedup vs baseline: 10.3675x; 1.0596x over previous
import jax
import jax.numpy as jnp
from jax import lax
from jax.experimental import pallas as pl
from jax.experimental.pallas import tpu as pltpu
from jax.experimental.pallas import tpu_sc as plsc

_N = 10000
_D = 128
_E = 320000
_NSUB = 16
_NCORE = 2
_EPW = _E // (_NCORE * _NSUB)
_RPS = _N // _NSUB


def _make_agg_resident(width, chunk):
    nchunk = _EPW // chunk
    assert nchunk * chunk == _EPW and nchunk % 2 == 0
    mesh = plsc.VectorSubcoreMesh(core_axis_name="c", subcore_axis_name="s")

    def body(ei, table, zeros, out, acc, sidx, didx, rows, g0, g1):
        c = lax.axis_index("c")
        s = lax.axis_index("s")
        w = c * _NSUB + s
        base = s * _RPS
        pltpu.async_copy(ei.at[0, w], sidx, g0)
        pltpu.async_copy(ei.at[1, w], didx, g1)
        pltpu.sync_copy(zeros.at[pl.ds(base, _RPS)], acc.at[pl.ds(base, _RPS)])
        pltpu.make_async_copy(ei.at[0, 0], sidx, g0).wait()
        pltpu.make_async_copy(ei.at[1, 0], didx, g1).wait()
        pltpu.async_copy(table.at[sidx.at[0]], rows.at[0], g0)
        pltpu.async_copy(table.at[sidx.at[1]], rows.at[1], g1)
        plsc.subcore_barrier()

        @pl.loop(0, nchunk, step=2)
        def _(j):
            pltpu.make_async_copy(
                table.at[pl.ds(0, chunk)], rows.at[0], g0).wait()
            pltpu.sync_copy(rows.at[0], acc.at[didx.at[j]], add=True)

            @pl.when(j + 2 < nchunk)
            def _():
                pltpu.async_copy(table.at[sidx.at[j + 2]], rows.at[0], g0)

            pltpu.make_async_copy(
                table.at[pl.ds(0, chunk)], rows.at[1], g1).wait()
            pltpu.sync_copy(rows.at[1], acc.at[didx.at[j + 1]], add=True)

            @pl.when(j + 3 < nchunk)
            def _():
                pltpu.async_copy(table.at[sidx.at[j + 3]], rows.at[1], g1)

        plsc.subcore_barrier()
        pltpu.sync_copy(acc.at[pl.ds(base, _RPS)],
                        out.at[c, pl.ds(base, _RPS)])

    return pl.kernel(
        body,
        out_type=jax.ShapeDtypeStruct((_NCORE, _N, width), jnp.float32),
        mesh=mesh,
        compiler_params=pltpu.CompilerParams(use_tc_tiling_on_sc=False),
        scratch_types=[
            pltpu.VMEM_SHARED((_N, width), jnp.float32),
            pltpu.VMEM((nchunk, chunk), jnp.int32),
            pltpu.VMEM((nchunk, chunk), jnp.int32),
            pltpu.VMEM((2, chunk, width), jnp.float32),
            pltpu.SemaphoreType.DMA,
            pltpu.SemaphoreType.DMA,
        ],
    )


def _make_agg_streamed(width, chunk):
    nchunk = _EPW // chunk
    assert nchunk * chunk == _EPW and nchunk % 2 == 0
    mesh = plsc.VectorSubcoreMesh(core_axis_name="c", subcore_axis_name="s")

    def body(ei, table, zeros, out, acc, sidx, didx, rows, g0, g1, i0, i1):
        c = lax.axis_index("c")
        s = lax.axis_index("s")
        w = c * _NSUB + s
        base = s * _RPS
        pltpu.async_copy(ei.at[0, w, 0], sidx.at[0], i0)
        pltpu.async_copy(ei.at[1, w, 0], didx.at[0], i0)
        pltpu.async_copy(ei.at[0, w, 1], sidx.at[1], i1)
        pltpu.async_copy(ei.at[1, w, 1], didx.at[1], i1)
        pltpu.sync_copy(zeros.at[pl.ds(base, _RPS)], acc.at[pl.ds(base, _RPS)])
        pltpu.make_async_copy(ei.at[0, 0, 0], sidx.at[0], i0).wait()
        pltpu.make_async_copy(ei.at[0, 0, 0], didx.at[0], i0).wait()
        pltpu.async_copy(table.at[sidx.at[0]], rows.at[0], g0)
        pltpu.make_async_copy(ei.at[0, 0, 0], sidx.at[1], i1).wait()
        pltpu.make_async_copy(ei.at[0, 0, 0], didx.at[1], i1).wait()
        pltpu.async_copy(table.at[sidx.at[1]], rows.at[1], g1)
        plsc.subcore_barrier()

        @pl.loop(0, nchunk, step=2)
        def _(j):
            pltpu.make_async_copy(
                table.at[pl.ds(0, chunk)], rows.at[0], g0).wait()

            @pl.when(j + 2 < nchunk)
            def _():
                pltpu.async_copy(ei.at[0, w, j + 2], sidx.at[0], i0)

            pltpu.sync_copy(rows.at[0], acc.at[didx.at[0]], add=True)

            @pl.when(j + 2 < nchunk)
            def _():
                pltpu.async_copy(ei.at[1, w, j + 2], didx.at[0], i0)

            pltpu.make_async_copy(
                table.at[pl.ds(0, chunk)], rows.at[1], g1).wait()

            @pl.when(j + 3 < nchunk)
            def _():
                pltpu.async_copy(ei.at[0, w, j + 3], sidx.at[1], i1)

            pltpu.sync_copy(rows.at[1], acc.at[didx.at[1]], add=True)

            @pl.when(j + 3 < nchunk)
            def _():
                pltpu.async_copy(ei.at[1, w, j + 3], didx.at[1], i1)

            @pl.when(j + 2 < nchunk)
            def _():
                pltpu.make_async_copy(ei.at[0, 0, 0], sidx.at[0], i0).wait()
                pltpu.make_async_copy(ei.at[0, 0, 0], didx.at[0], i0).wait()
                pltpu.async_copy(table.at[sidx.at[0]], rows.at[0], g0)

            @pl.when(j + 3 < nchunk)
            def _():
                pltpu.make_async_copy(ei.at[0, 0, 0], sidx.at[1], i1).wait()
                pltpu.make_async_copy(ei.at[0, 0, 0], didx.at[1], i1).wait()
                pltpu.async_copy(table.at[sidx.at[1]], rows.at[1], g1)

        plsc.subcore_barrier()
        pltpu.sync_copy(acc.at[pl.ds(base, _RPS)],
                        out.at[c, pl.ds(base, _RPS)])

    return pl.kernel(
        body,
        out_type=jax.ShapeDtypeStruct((_NCORE, _N, width), jnp.float32),
        mesh=mesh,
        compiler_params=pltpu.CompilerParams(use_tc_tiling_on_sc=False),
        scratch_types=[
            pltpu.VMEM_SHARED((_N, width), jnp.float32),
            pltpu.VMEM((2, chunk), jnp.int32),
            pltpu.VMEM((2, chunk), jnp.int32),
            pltpu.VMEM((2, chunk, width), jnp.float32),
            pltpu.SemaphoreType.DMA,
            pltpu.SemaphoreType.DMA,
            pltpu.SemaphoreType.DMA,
            pltpu.SemaphoreType.DMA,
        ],
    )


_CHUNK_L1 = 125
_CHUNK = 100
_agg_l1 = _make_agg_streamed(_D + 16, _CHUNK_L1)
_agg = _make_agg_resident(_D, _CHUNK)


def _layer1_body(p_ref, x_ref, wl_ref, bl_ref, wr_ref, h_ref, inv_ref):
    msum = p_ref[0, :, :_D] + p_ref[1, :, :_D]
    cnt = p_ref[0, :, _D:_D + 1] + p_ref[1, :, _D:_D + 1]
    inv = 1.0 / jnp.maximum(cnt, 1.0)
    mean = msum * inv
    h = jnp.dot(mean, wl_ref[...], preferred_element_type=jnp.float32)
    h = h + bl_ref[...] + jnp.dot(x_ref[...], wr_ref[...],
                                  preferred_element_type=jnp.float32)
    h_ref[...] = jnp.maximum(h, 0.0)
    inv_ref[...] = inv


_layer1_tc = pl.pallas_call(
    _layer1_body,
    out_shape=(
        jax.ShapeDtypeStruct((_N, _D), jnp.float32),
        jax.ShapeDtypeStruct((_N, 1), jnp.float32),
    ),
)


def _make_layer23(relu):
    def body(p_ref, inv_ref, x_ref, wl_ref, bl_ref, wr_ref, o_ref):
        mean = (p_ref[0] + p_ref[1]) * inv_ref[...]
        h = jnp.dot(mean, wl_ref[...], preferred_element_type=jnp.float32)
        h = h + bl_ref[...] + jnp.dot(x_ref[...], wr_ref[...],
                                      preferred_element_type=jnp.float32)
        o_ref[...] = jnp.maximum(h, 0.0) if relu else h

    return pl.pallas_call(
        body, out_shape=jax.ShapeDtypeStruct((_N, _D), jnp.float32))


_layer2_tc = _make_layer23(True)
_layer3_tc = _make_layer23(False)


def kernel(x, edge_index, Wl1, bl1, Wr1, Wl2, bl2, Wr2, Wl3, bl3, Wr3):
    ei = edge_index.astype(jnp.int32)
    ei_l1 = ei.reshape(2, _NCORE * _NSUB, _EPW // _CHUNK_L1, _CHUNK_L1)
    ei_23 = ei.reshape(2, _NCORE * _NSUB, _EPW // _CHUNK, _CHUNK)
    xa = jnp.concatenate(
        [x, jnp.ones((_N, 1), jnp.float32), jnp.zeros((_N, 15), jnp.float32)],
        axis=1)
    z_l1 = jnp.zeros((_N, _D + 16), jnp.float32)
    z = jnp.zeros((_N, _D), jnp.float32)
    bl1r = bl1.reshape(1, _D)
    bl2r = bl2.reshape(1, _D)
    bl3r = bl3.reshape(1, _D)

    p1 = _agg_l1(ei_l1, xa, z_l1)
    h1, inv = _layer1_tc(p1, x, Wl1, bl1r, Wr1)
    p2 = _agg(ei_23, h1, z)
    h2 = _layer2_tc(p2, inv, h1, Wl2, bl2r, Wr2)
    p3 = _agg(ei_23, h2, z)
    return _layer3_tc(p3, inv, h2, Wl3, bl3r, Wr3)


# speedup vs baseline: 10.9905x; 1.0601x over previous
import jax
import jax.numpy as jnp
from jax import lax
from jax.experimental import pallas as pl
from jax.experimental.pallas import tpu as pltpu
from jax.experimental.pallas import tpu_sc as plsc

_N = 10000
_D = 128
_E = 320000
_NSUB = 16
_NCORE = 2
_EPW = _E // (_NCORE * _NSUB)
_RPS = _N // _NSUB


def _make_agg_resident(width, chunk, halves=1):
    nchunk = _EPW // chunk
    nres = nchunk // halves
    assert nchunk * chunk == _EPW and nres * halves == nchunk and nres % 2 == 0
    mesh = plsc.VectorSubcoreMesh(core_axis_name="c", subcore_axis_name="s")

    def body(ei, table, zeros, out, acc, sidx, didx, rows, g0, g1):
        c = lax.axis_index("c")
        s = lax.axis_index("s")
        w = c * _NSUB + s
        base = s * _RPS
        pltpu.async_copy(ei.at[0, w, pl.ds(0, nres)], sidx, g0)
        pltpu.async_copy(ei.at[1, w, pl.ds(0, nres)], didx, g1)
        pltpu.sync_copy(zeros.at[pl.ds(base, _RPS)], acc.at[pl.ds(base, _RPS)])
        pltpu.make_async_copy(ei.at[0, 0, pl.ds(0, nres)], sidx, g0).wait()
        pltpu.make_async_copy(ei.at[1, 0, pl.ds(0, nres)], didx, g1).wait()
        pltpu.async_copy(table.at[sidx.at[0]], rows.at[0], g0)
        pltpu.async_copy(table.at[sidx.at[1]], rows.at[1], g1)
        plsc.subcore_barrier()

        for h in range(halves):
            if h > 0:
                pltpu.sync_copy(ei.at[0, w, pl.ds(h * nres, nres)], sidx)
                pltpu.sync_copy(ei.at[1, w, pl.ds(h * nres, nres)], didx)
                pltpu.async_copy(table.at[sidx.at[0]], rows.at[0], g0)
                pltpu.async_copy(table.at[sidx.at[1]], rows.at[1], g1)

            @pl.loop(0, nres, step=2)
            def _(j):
                pltpu.make_async_copy(
                    table.at[pl.ds(0, chunk)], rows.at[0], g0).wait()
                pltpu.sync_copy(rows.at[0], acc.at[didx.at[j]], add=True)

                @pl.when(j + 2 < nres)
                def _():
                    pltpu.async_copy(table.at[sidx.at[j + 2]], rows.at[0], g0)

                pltpu.make_async_copy(
                    table.at[pl.ds(0, chunk)], rows.at[1], g1).wait()
                pltpu.sync_copy(rows.at[1], acc.at[didx.at[j + 1]], add=True)

                @pl.when(j + 3 < nres)
                def _():
                    pltpu.async_copy(table.at[sidx.at[j + 3]], rows.at[1], g1)

        plsc.subcore_barrier()
        pltpu.sync_copy(acc.at[pl.ds(base, _RPS)],
                        out.at[c, pl.ds(base, _RPS)])

    return pl.kernel(
        body,
        out_type=jax.ShapeDtypeStruct((_NCORE, _N, width), jnp.float32),
        mesh=mesh,
        compiler_params=pltpu.CompilerParams(use_tc_tiling_on_sc=False),
        scratch_types=[
            pltpu.VMEM_SHARED((_N, width), jnp.float32),
            pltpu.VMEM((nres, chunk), jnp.int32),
            pltpu.VMEM((nres, chunk), jnp.int32),
            pltpu.VMEM((2, chunk, width), jnp.float32),
            pltpu.SemaphoreType.DMA,
            pltpu.SemaphoreType.DMA,
        ],
    )


_CHUNK = 100
_agg_l1 = _make_agg_resident(_D + 16, _CHUNK, halves=2)
_agg = _make_agg_resident(_D, _CHUNK)


def _layer1_body(p_ref, x_ref, wl_ref, bl_ref, wr_ref, h_ref, inv_ref):
    msum = p_ref[0, :, :_D] + p_ref[1, :, :_D]
    cnt = p_ref[0, :, _D:_D + 1] + p_ref[1, :, _D:_D + 1]
    inv = 1.0 / jnp.maximum(cnt, 1.0)
    mean = msum * inv
    h = jnp.dot(mean, wl_ref[...], preferred_element_type=jnp.float32)
    h = h + bl_ref[...] + jnp.dot(x_ref[...], wr_ref[...],
                                  preferred_element_type=jnp.float32)
    h_ref[...] = jnp.maximum(h, 0.0)
    inv_ref[...] = inv


_layer1_tc = pl.pallas_call(
    _layer1_body,
    out_shape=(
        jax.ShapeDtypeStruct((_N, _D), jnp.float32),
        jax.ShapeDtypeStruct((_N, 1), jnp.float32),
    ),
)


def _make_layer23(relu):
    def body(p_ref, inv_ref, x_ref, wl_ref, bl_ref, wr_ref, o_ref):
        mean = (p_ref[0] + p_ref[1]) * inv_ref[...]
        h = jnp.dot(mean, wl_ref[...], preferred_element_type=jnp.float32)
        h = h + bl_ref[...] + jnp.dot(x_ref[...], wr_ref[...],
                                      preferred_element_type=jnp.float32)
        o_ref[...] = jnp.maximum(h, 0.0) if relu else h

    return pl.pallas_call(
        body, out_shape=jax.ShapeDtypeStruct((_N, _D), jnp.float32))


_layer2_tc = _make_layer23(True)
_layer3_tc = _make_layer23(False)


def kernel(x, edge_index, Wl1, bl1, Wr1, Wl2, bl2, Wr2, Wl3, bl3, Wr3):
    ei = edge_index.astype(jnp.int32)
    eir = ei.reshape(2, _NCORE * _NSUB, _EPW // _CHUNK, _CHUNK)
    xa = jnp.concatenate(
        [x, jnp.ones((_N, 1), jnp.float32), jnp.zeros((_N, 15), jnp.float32)],
        axis=1)
    z_l1 = jnp.zeros((_N, _D + 16), jnp.float32)
    z = jnp.zeros((_N, _D), jnp.float32)
    bl1r = bl1.reshape(1, _D)
    bl2r = bl2.reshape(1, _D)
    bl3r = bl3.reshape(1, _D)

    p1 = _agg_l1(eir, xa, z_l1)
    h1, inv = _layer1_tc(p1, x, Wl1, bl1r, Wr1)
    p2 = _agg(eir, h1, z)
    h2 = _layer2_tc(p2, inv, h1, Wl2, bl2r, Wr2)
    p3 = _agg(eir, h2, z)
    return _layer3_tc(p3, inv, h2, Wl3, bl3r, Wr3)
